# baseline (speedup 1.0000x reference)
import functools

import numpy as np
import jax
import jax.numpy as jnp
from jax import lax
from jax.experimental import pallas as pl
from jax.experimental.pallas import tpu as pltpu
from jax.experimental.pallas import tpu_sc as plsc

PC_RANGE = [-51.2, -51.2, -5.0, 51.2, 51.2, 3.0]
IMG_H, IMG_W = 256, 704
LVL_HW = ((32, 88), (16, 44), (8, 22), (4, 11))
NV = 6
NL = 4
NF = 8
C = 128
NQ = 2500
NZ = 4
QP = NQ * NZ
QPP = 10240
NJ = NV * NL * 4
NJR = NV * 2
DIDX = 64

BLK_A = 512

NW = 32
PTS_W = QPP // NW
CH = 32
KG = 16
ND = KG * NJR // DIDX
NGC = CH // KG

_LVL_BASE = []
_acc = 0
for _h, _w in LVL_HW[:2]:
    _LVL_BASE.append(_acc)
    _acc += NV * (_h - 1) * (_w - 1)
_LVL_BASE += [0, 0]
N_ROWS = _acc

_F32 = jnp.float32
_HI = lax.Precision.HIGHEST


def _np_lane_consts():
    wscale = np.zeros((1, 24), np.float32)
    hscale = np.zeros((1, 24), np.float32)
    wm1 = np.zeros((1, 24), np.float32)
    hm1 = np.zeros((1, 24), np.float32)
    wl = np.zeros((1, 24), np.float32)
    base = np.zeros((1, 24), np.float32)
    for v in range(NV):
        for l in range(NL):
            h, w = LVL_HW[l]
            j = l * 6 + v
            wscale[0, j] = w / IMG_W
            hscale[0, j] = h / IMG_H
            wm1[0, j] = w - 1
            hm1[0, j] = h - 1
            wl[0, j] = w - 1
            base[0, j] = _LVL_BASE[l] + v * (h - 1) * (w - 1)
    return wscale, hscale, wm1, hm1, wl, base


_WSCALE, _HSCALE, _WM1, _HM1, _WL, _BASE = _np_lane_consts()


def _np_pe_consts():
    p48 = np.zeros((3, 48), np.float32)
    ph48 = np.zeros((1, 48), np.float32)
    for d in range(3):
        for s in range(16):
            col = d * 16 + s
            f = s % 8
            p48[d, col] = (2.0 ** f) * np.pi
            ph48[0, col] = 0.0 if s < 8 else np.pi / 2.0
    return p48, ph48


_P48, _PH48 = _np_pe_consts()
_SCALE3 = np.array([[PC_RANGE[3] - PC_RANGE[0],
                     PC_RANGE[4] - PC_RANGE[1],
                     PC_RANGE[5] - PC_RANGE[2]]], np.float32)
_OFF3 = np.array([[PC_RANGE[0], PC_RANGE[1], PC_RANGE[2]]], np.float32)

_LC = np.zeros((8, 24), np.float32)
_LC[0:1] = _WSCALE
_LC[1:2] = _HSCALE
_LC[2:3] = _WM1
_LC[3:4] = _HM1
_LC[4:5] = _WL
_LC[5:6] = _BASE
_LC[6, 0:3] = _SCALE3[0]
_LC[7, 0:3] = _OFF3[0]


def _precomp_body(pos_ref, a0_ref, a1_ref, a2_ref,
                  pw1_ref, pb1_ref, pw2_ref, pb2_ref, ww_ref, wb_ref,
                  p48_ref, ph48_ref, lc_ref, f2_ref, f3_ref,
                  idx_ref, wgt_ref, pe_ref):
    pos = pos_ref[...]
    ang = lax.dot_general(pos, p48_ref[...],
                          (((1,), (0,)), ((), ())), precision=_HI)
    enc = jnp.sin(ang + ph48_ref[...])
    h = jnp.maximum(
        lax.dot_general(enc, pw1_ref[...], (((1,), (0,)), ((), ())),
                        precision=None) + pb1_ref[...], 0.0)
    pe = lax.dot_general(h, pw2_ref[...], (((1,), (0,)), ((), ())),
                         precision=None) + pb2_ref[...]
    logits = lax.dot_general(pe, ww_ref[...], (((1,), (0,)), ((), ())),
                             precision=None) + wb_ref[...]
    m = jnp.max(logits, axis=1, keepdims=True)
    e = jnp.exp(logits - m)
    sw = e / jnp.sum(e, axis=1, keepdims=True)
    sw24 = jnp.repeat(sw, NV, axis=1)
    lc = lc_ref[...]
    rp = pos * lc[6:7, 0:3] + lc[7:8, 0:3]
    rph = jnp.concatenate(
        [rp, jnp.ones((rp.shape[0], 1), _F32)], axis=1)
    cx = lax.dot_general(rph, a0_ref[...], (((1,), (0,)), ((), ())),
                         precision=None)
    cy = lax.dot_general(rph, a1_ref[...], (((1,), (0,)), ((), ())),
                         precision=None)
    cz = lax.dot_general(rph, a2_ref[...], (((1,), (0,)), ((), ())),
                         precision=None)
    homo = jnp.maximum(cz, 1e-5)
    xs = cx / homo * lc[0:1, :] - 0.5
    ys = cy / homo * lc[1:2, :] - 0.5
    x0 = jnp.floor(xs)
    y0 = jnp.floor(ys)
    fx = xs - x0
    fy = ys - y0
    wm1 = lc[2:3, :]
    hm1 = lc[3:4, :]
    vx0 = ((x0 >= 0.0) & (x0 <= wm1)).astype(_F32)
    vx1 = ((x0 + 1.0 >= 0.0) & (x0 + 1.0 <= wm1)).astype(_F32)
    vy0 = ((y0 >= 0.0) & (y0 <= hm1)).astype(_F32)
    vy1 = ((y0 + 1.0 >= 0.0) & (y0 + 1.0 <= hm1)).astype(_F32)
    xc0 = jnp.clip(x0, 0.0, wm1)
    xc1 = jnp.clip(x0 + 1.0, 0.0, wm1)
    yc0 = jnp.clip(y0, 0.0, hm1)
    yc1 = jnp.clip(y0 + 1.0, 0.0, hm1)
    xl = jnp.clip(x0, 0.0, wm1 - 1.0)
    yl = jnp.clip(y0, 0.0, hm1 - 1.0)
    wx0 = (1.0 - fx) * vx0
    wx1 = fx * vx1
    wy0 = (1.0 - fy) * vy0
    wy1 = fy * vy1
    whx0 = (wx0 * (xc0 == xl).astype(_F32)
            + wx1 * (xc1 == xl).astype(_F32))
    whx1 = (wx0 * (xc0 == xl + 1.0).astype(_F32)
            + wx1 * (xc1 == xl + 1.0).astype(_F32))
    why0 = (wy0 * (yc0 == yl).astype(_F32)
            + wy1 * (yc1 == yl).astype(_F32))
    why1 = (wy0 * (yc0 == yl + 1.0).astype(_F32)
            + wy1 * (yc1 == yl + 1.0).astype(_F32))
    stride = lc[4:5, :]
    pbase = lc[5:6, :]
    idx_ref[...] = (pbase + yl * stride + xl).astype(jnp.int32)[:, :NJR]
    wgt_ref[...] = jnp.concatenate(
        [why0 * whx0 * sw24, why0 * whx1 * sw24,
         why1 * whx0 * sw24, why1 * whx1 * sw24], axis=1)
    wimg = stride + 1.0
    pix00 = yc0 * wimg + xc0
    pix01 = yc0 * wimg + xc1
    pix10 = yc1 * wimg + xc0
    pix11 = yc1 * wimg + xc1
    cw00 = wx0 * wy0 * sw24
    cw01 = wx1 * wy0 * sw24
    cw10 = wx0 * wy1 * sw24
    cw11 = wx1 * wy1 * sw24
    s23 = jnp.zeros((pos.shape[0], C), _F32)
    for li, f_ref in ((2, f2_ref), (3, f3_ref)):
        hw = LVL_HW[li][0] * LVL_HW[li][1]
        iot = lax.broadcasted_iota(jnp.int32, (1, hw), 1).astype(_F32)
        ohs = []
        for v in range(NV):
            q = li * 6 + v
            oh = (cw00[:, q:q + 1] * (pix00[:, q:q + 1] == iot).astype(_F32)
                  + cw01[:, q:q + 1] * (pix01[:, q:q + 1] == iot).astype(_F32)
                  + cw10[:, q:q + 1] * (pix10[:, q:q + 1] == iot).astype(_F32)
                  + cw11[:, q:q + 1] * (pix11[:, q:q + 1] == iot).astype(_F32))
            ohs.append(oh)
        ohcat = jnp.concatenate(ohs, axis=1)
        s23 = s23 + lax.dot_general(ohcat, f_ref[...],
                                    (((1,), (0,)), ((), ())), precision=_HI)
    pe_ref[...] = pe + s23


def _run_precomp(pos_p, a0, a1, a2, pe_w1, pe_b1, pe_w2, pe_b2, wgt_w, wgt_b,
                 f2t, f3t):
    n_blk = QPP // BLK_A
    full = lambda shape: pl.BlockSpec(shape, lambda i: (0,) * len(shape))
    return pl.pallas_call(
        _precomp_body,
        grid=(n_blk,),
        in_specs=[
            pl.BlockSpec((BLK_A, 3), lambda i: (i, 0)),
            full((4, 24)), full((4, 24)), full((4, 24)),
            full((48, 256)), full((1, 256)),
            full((256, 128)), full((1, 128)),
            full((128, 4)), full((1, 4)),
            full((3, 48)), full((1, 48)), full((8, 24)),
            full((6 * 8 * 22, C)), full((6 * 4 * 11, C)),
        ],
        out_specs=[
            pl.BlockSpec((BLK_A, NJR), lambda i: (i, 0)),
            pl.BlockSpec((BLK_A, NJ), lambda i: (i, 0)),
            pl.BlockSpec((BLK_A, C), lambda i: (i, 0)),
        ],
        out_shape=[
            jax.ShapeDtypeStruct((QPP, NJR), jnp.int32),
            jax.ShapeDtypeStruct((QPP, NJ), _F32),
            jax.ShapeDtypeStruct((QPP, C), _F32),
        ],
    )(pos_p, a0, a1, a2, pe_w1, pe_b1, pe_w2, pe_b2, wgt_w, wgt_b,
      jnp.asarray(_P48), jnp.asarray(_PH48), jnp.asarray(_LC), f2t, f3t)


def _bcast_lane(vec, t):
    dn = lax.GatherDimensionNumbers(
        offset_dims=(), collapsed_slice_dims=(0,), start_index_map=(0,))
    return lax.gather(vec, jnp.full((16, 1), t, jnp.int32), dn, (1,),
                      mode=lax.GatherScatterMode.PROMISE_IN_BOUNDS)


def _sc_body(idx_hbm, wgt_hbm, pe_hbm, table_hbm, out_hbm,
             idx_v, wgt_v, acc_v, rows_v, *sems):
    cc = lax.axis_index("c")
    ss = lax.axis_index("s")
    wid = ss * 2 + cc
    base = wid * PTS_W
    hi_mask = jnp.full((16,), np.int32(-65536), jnp.int32)

    def mac_point(p, b, pp):
        accs = [acc_v[p, pl.ds(cch * 16, 16)] for cch in range(8)]
        wvecs = [wgt_v[p, pl.ds(k * 16, 16)] for k in range(NJ // 16)]
        for j in range(NJR):
            ws = []
            for cslot in range(4):
                lw = cslot * 24 + j
                ws.append(_bcast_lane(wvecs[lw // 16], lw % 16))
            flat = pp * NJR + j
            d = flat // DIDX
            s = flat - d * DIDX
            for c4 in range(4):
                lo = accs[2 * c4]
                hi = accs[2 * c4 + 1]
                for cslot in range(4):
                    pw = rows_v[b, d, s, pl.ds(cslot * 64 + c4 * 16, 16)]
                    ea = lax.bitcast_convert_type(pw << 16, _F32)
                    eb = lax.bitcast_convert_type(pw & hi_mask, _F32)
                    lo = lo + ws[cslot] * ea
                    hi = hi + ws[cslot] * eb
                accs[2 * c4] = lo
                accs[2 * c4 + 1] = hi
        for cch in range(8):
            acc_v[p, pl.ds(cch * 16, 16)] = accs[cch]

    def fire(g, b):
        for t in range(ND):
            pltpu.async_copy(table_hbm.at[idx_v.at[g * ND + t, 0]],
                             rows_v.at[b, t], sems[b])

    def drain(g, b):
        for t in range(ND):
            pltpu.make_async_copy(table_hbm.at[idx_v.at[g * ND + t, 0]],
                                  rows_v.at[b, t], sems[b]).wait()

    def mac_group(g, b):
        def body(pp, carry):
            mac_point(g * KG + pp, b, pp)
            return carry

        lax.fori_loop(0, KG, body, 0)

    IR = CH * NJR // DIDX

    def chunk_body(ci, carry):
        cb = base + ci * CH
        pltpu.sync_copy(idx_hbm.at[pl.ds((wid * PTS_W // CH + ci) * IR, IR)],
                        idx_v)
        pltpu.sync_copy(wgt_hbm.at[pl.ds(cb, CH)], wgt_v)
        pltpu.sync_copy(pe_hbm.at[pl.ds(cb, CH)], acc_v)
        fire(0, 0)
        fire(1, 1)

        def gpair(i, carry2):
            g0 = 2 * i
            drain(g0, 0)
            mac_group(g0, 0)

            @pl.when(g0 + 2 < NGC)
            def _():
                fire(g0 + 2, 0)

            drain(g0 + 1, 1)
            mac_group(g0 + 1, 1)

            @pl.when(g0 + 3 < NGC)
            def _():
                fire(g0 + 3, 1)

            return carry2

        lax.fori_loop(0, NGC // 2, gpair, 0)
        pltpu.sync_copy(acc_v, out_hbm.at[pl.ds(cb, CH)])
        return carry

    lax.fori_loop(0, PTS_W // CH, chunk_body, 0)


def _run_sc_gather(idx, wgt, pe, table):
    mesh = plsc.VectorSubcoreMesh(core_axis_name="c", subcore_axis_name="s")
    fn = functools.partial(
        pl.kernel,
        mesh=mesh,
        out_type=jax.ShapeDtypeStruct((QPP, C), _F32),
        scratch_types=[
            pltpu.VMEM((CH * NJR // DIDX, 1, DIDX), jnp.int32),
            pltpu.VMEM((CH, NJ), _F32),
            pltpu.VMEM((CH, C), _F32),
            pltpu.VMEM((2, ND, DIDX, 2 * C), jnp.int32),
        ] + [pltpu.SemaphoreType.DMA] * 2,
    )(_sc_body)
    return fn(idx, wgt, pe, table)


def _hmlp_body(x_ref, w1_ref, b1_ref, w2_ref, b2_ref, w3_ref, b3_ref,
               w4_ref, b4_ref, o_ref):
    x = x_ref[...]
    h = jnp.maximum(
        lax.dot_general(x, w1_ref[...], (((1,), (0,)), ((), ())),
                        precision=None) + b1_ref[...], 0.0)
    h = jnp.maximum(
        lax.dot_general(h, w2_ref[...], (((1,), (0,)), ((), ())),
                        precision=None) + b2_ref[...], 0.0)
    h = jnp.maximum(
        lax.dot_general(h, w3_ref[...], (((1,), (0,)), ((), ())),
                        precision=None) + b3_ref[...], 0.0)
    o_ref[...] = lax.dot_general(h, w4_ref[...], (((1,), (0,)), ((), ())),
                                 precision=None) + b4_ref[...]


NQP = 2560
BLK_C = 512


def _run_hmlp(x, w1, b1, w2, b2, w3, b3, w4, b4):
    full = lambda shape: pl.BlockSpec(shape, lambda i: (0,) * len(shape))
    return pl.pallas_call(
        _hmlp_body,
        grid=(NQP // BLK_C,),
        in_specs=[
            pl.BlockSpec((BLK_C, NZ * C), lambda i: (i, 0)),
            full((NZ * C, 1024)), full((1, 1024)),
            full((1024, 1024)), full((1, 1024)),
            full((1024, 1024)), full((1, 1024)),
            full((1024, C)), full((1, C)),
        ],
        out_specs=pl.BlockSpec((BLK_C, C), lambda i: (i, 0)),
        out_shape=jax.ShapeDtypeStruct((NQP, C), _F32),
    )(x, w1, b1, w2, b2, w3, b3, w4, b4)


def _flatten_feats(feat0, feat1, feat2, feat3):
    rows = []
    for f in (feat0, feat1):
        g = jnp.transpose(f[0], (0, 2, 3, 1))
        v, h, w, c = g.shape
        pr = jnp.concatenate(
            [g[:, :-1, :-1, :], g[:, :-1, 1:, :],
             g[:, 1:, :-1, :], g[:, 1:, 1:, :]], axis=-1)
        rows.append(pr.reshape(v * (h - 1) * (w - 1), 4 * c))
    t = jnp.concatenate(rows, axis=0)
    t32 = t.reshape(N_ROWS, 16, 32)
    t = jnp.stack([t32[..., :16], t32[..., 16:]], axis=-1).reshape(N_ROWS, 4 * C)
    t16 = t.astype(jnp.bfloat16).reshape(N_ROWS, 2 * C, 2)
    return jax.lax.bitcast_convert_type(t16, jnp.int32)


def kernel(feat0, feat1, feat2, feat3, reference_points, lidar2img,
           pe_w1, pe_b1, pe_w2, pe_b2, wgt_w, wgt_b,
           hm_w1, hm_b1, hm_w2, hm_b2, hm_w3, hm_b3, hm_w4, hm_b4):
    pos = jnp.transpose(reference_points[0], (1, 0, 2)).reshape(QP, 3)
    pos_p = jnp.pad(pos, ((0, QPP - QP), (0, 0)))
    m = lidar2img[0, 0]
    a0 = jnp.tile(m[:, 0, :], (NL, 1)).T
    a1 = jnp.tile(m[:, 1, :], (NL, 1)).T
    a2 = jnp.tile(m[:, 2, :], (NL, 1)).T
    table = _flatten_feats(feat0, feat1, feat2, feat3)
    f2t = jnp.transpose(feat2[0], (0, 2, 3, 1)).reshape(6 * 8 * 22, C)
    f3t = jnp.transpose(feat3[0], (0, 2, 3, 1)).reshape(6 * 4 * 11, C)

    idx, wgt, pe = _run_precomp(
        pos_p, a0, a1, a2,
        pe_w1, pe_b1.reshape(1, -1), pe_w2, pe_b2.reshape(1, -1),
        wgt_w, wgt_b.reshape(1, -1), f2t, f3t)

    idx = idx.reshape(QPP * NJR // DIDX, 1, DIDX)
    sampled = _run_sc_gather(idx, wgt, pe, table)

    x = sampled[:QP].reshape(NQ, NZ * C)
    x = jnp.pad(x, ((0, NQP - NQ), (0, 0)))
    out = _run_hmlp(
        x, hm_w1, hm_b1.reshape(1, -1), hm_w2, hm_b2.reshape(1, -1),
        hm_w3, hm_b3.reshape(1, -1), hm_w4, hm_b4.reshape(1, -1))
    return out[:NQ].reshape(1, NQ, C)


# speedup vs baseline: 1.0511x; 1.0511x over previous
import functools

import numpy as np
import jax
import jax.numpy as jnp
from jax import lax
from jax.experimental import pallas as pl
from jax.experimental.pallas import tpu as pltpu
from jax.experimental.pallas import tpu_sc as plsc

PC_RANGE = [-51.2, -51.2, -5.0, 51.2, 51.2, 3.0]
IMG_H, IMG_W = 256, 704
LVL_HW = ((32, 88), (16, 44), (8, 22), (4, 11))
NV = 6
NL = 4
NF = 8
C = 128
NQ = 2500
NZ = 4
QP = NQ * NZ
QPP = 10240
NJ = NV * NL * 4
NJR = NV * 2
DIDX = 64

BLK_A = 512

NW = 32
PTS_W = QPP // NW
CH = 32
KG = 16
ND = KG * NJR // DIDX
NGC = CH // KG

_LVL_BASE = []
_acc = 0
for _h, _w in LVL_HW[:2]:
    _LVL_BASE.append(_acc)
    _acc += NV * (_h - 1) * (_w - 1)
_LVL_BASE += [0, 0]
N_ROWS = _acc

_F32 = jnp.float32
_HI = lax.Precision.HIGHEST


def _np_lane_consts():
    wscale = np.zeros((1, 24), np.float32)
    hscale = np.zeros((1, 24), np.float32)
    wm1 = np.zeros((1, 24), np.float32)
    hm1 = np.zeros((1, 24), np.float32)
    wl = np.zeros((1, 24), np.float32)
    base = np.zeros((1, 24), np.float32)
    for v in range(NV):
        for l in range(NL):
            h, w = LVL_HW[l]
            j = l * 6 + v
            wscale[0, j] = w / IMG_W
            hscale[0, j] = h / IMG_H
            wm1[0, j] = w - 1
            hm1[0, j] = h - 1
            wl[0, j] = w - 1
            base[0, j] = _LVL_BASE[l] + v * (h - 1) * (w - 1)
    return wscale, hscale, wm1, hm1, wl, base


_WSCALE, _HSCALE, _WM1, _HM1, _WL, _BASE = _np_lane_consts()


def _np_pe_consts():
    p48 = np.zeros((3, 48), np.float32)
    ph48 = np.zeros((1, 48), np.float32)
    for d in range(3):
        for s in range(16):
            col = d * 16 + s
            f = s % 8
            p48[d, col] = (2.0 ** f) * np.pi
            ph48[0, col] = 0.0 if s < 8 else np.pi / 2.0
    return p48, ph48


_P48, _PH48 = _np_pe_consts()
_SCALE3 = np.array([[PC_RANGE[3] - PC_RANGE[0],
                     PC_RANGE[4] - PC_RANGE[1],
                     PC_RANGE[5] - PC_RANGE[2]]], np.float32)
_OFF3 = np.array([[PC_RANGE[0], PC_RANGE[1], PC_RANGE[2]]], np.float32)

_LC = np.zeros((8, 24), np.float32)
_LC[0:1] = _WSCALE
_LC[1:2] = _HSCALE
_LC[2:3] = _WM1
_LC[3:4] = _HM1
_LC[4:5] = _WL
_LC[5:6] = _BASE
_LC[6, 0:3] = _SCALE3[0]
_LC[7, 0:3] = _OFF3[0]


def _precomp_body(pos_ref, a0_ref, a1_ref, a2_ref,
                  pw1_ref, pb1_ref, pw2_ref, pb2_ref, ww_ref, wb_ref,
                  p48_ref, ph48_ref, lc_ref, f2_ref, f3_ref,
                  idx_ref, wgt_ref, pe_ref):
    pos = pos_ref[...]
    ang = lax.dot_general(pos, p48_ref[...],
                          (((1,), (0,)), ((), ())), precision=_HI)
    enc = jnp.sin(ang + ph48_ref[...])
    h = jnp.maximum(
        lax.dot_general(enc, pw1_ref[...], (((1,), (0,)), ((), ())),
                        precision=None) + pb1_ref[...], 0.0)
    pe = lax.dot_general(h, pw2_ref[...], (((1,), (0,)), ((), ())),
                         precision=None) + pb2_ref[...]
    logits = lax.dot_general(pe, ww_ref[...], (((1,), (0,)), ((), ())),
                             precision=None) + wb_ref[...]
    m = jnp.max(logits, axis=1, keepdims=True)
    e = jnp.exp(logits - m)
    sw = e / jnp.sum(e, axis=1, keepdims=True)
    sw24 = jnp.repeat(sw, NV, axis=1)
    lc = lc_ref[...]
    rp = pos * lc[6:7, 0:3] + lc[7:8, 0:3]
    rph = jnp.concatenate(
        [rp, jnp.ones((rp.shape[0], 1), _F32)], axis=1)
    cx = lax.dot_general(rph, a0_ref[...], (((1,), (0,)), ((), ())),
                         precision=None)
    cy = lax.dot_general(rph, a1_ref[...], (((1,), (0,)), ((), ())),
                         precision=None)
    cz = lax.dot_general(rph, a2_ref[...], (((1,), (0,)), ((), ())),
                         precision=None)
    homo = jnp.maximum(cz, 1e-5)
    xs = cx / homo * lc[0:1, :] - 0.5
    ys = cy / homo * lc[1:2, :] - 0.5
    x0 = jnp.floor(xs)
    y0 = jnp.floor(ys)
    fx = xs - x0
    fy = ys - y0
    wm1 = lc[2:3, :]
    hm1 = lc[3:4, :]
    vx0 = ((x0 >= 0.0) & (x0 <= wm1)).astype(_F32)
    vx1 = ((x0 + 1.0 >= 0.0) & (x0 + 1.0 <= wm1)).astype(_F32)
    vy0 = ((y0 >= 0.0) & (y0 <= hm1)).astype(_F32)
    vy1 = ((y0 + 1.0 >= 0.0) & (y0 + 1.0 <= hm1)).astype(_F32)
    xc0 = jnp.clip(x0, 0.0, wm1)
    xc1 = jnp.clip(x0 + 1.0, 0.0, wm1)
    yc0 = jnp.clip(y0, 0.0, hm1)
    yc1 = jnp.clip(y0 + 1.0, 0.0, hm1)
    xl = jnp.clip(x0, 0.0, wm1 - 1.0)
    yl = jnp.clip(y0, 0.0, hm1 - 1.0)
    wx0 = (1.0 - fx) * vx0
    wx1 = fx * vx1
    wy0 = (1.0 - fy) * vy0
    wy1 = fy * vy1
    whx0 = (wx0 * (xc0 == xl).astype(_F32)
            + wx1 * (xc1 == xl).astype(_F32))
    whx1 = (wx0 * (xc0 == xl + 1.0).astype(_F32)
            + wx1 * (xc1 == xl + 1.0).astype(_F32))
    why0 = (wy0 * (yc0 == yl).astype(_F32)
            + wy1 * (yc1 == yl).astype(_F32))
    why1 = (wy0 * (yc0 == yl + 1.0).astype(_F32)
            + wy1 * (yc1 == yl + 1.0).astype(_F32))
    stride = lc[4:5, :]
    pbase = lc[5:6, :]
    idx_ref[...] = (pbase + yl * stride + xl).astype(jnp.int32)[:, :NJR]
    wgt_ref[...] = jnp.concatenate(
        [why0 * whx0 * sw24, why0 * whx1 * sw24,
         why1 * whx0 * sw24, why1 * whx1 * sw24], axis=1)
    wimg = stride + 1.0
    pix00 = yc0 * wimg + xc0
    pix01 = yc0 * wimg + xc1
    pix10 = yc1 * wimg + xc0
    pix11 = yc1 * wimg + xc1
    cw00 = wx0 * wy0 * sw24
    cw01 = wx1 * wy0 * sw24
    cw10 = wx0 * wy1 * sw24
    cw11 = wx1 * wy1 * sw24
    s23 = jnp.zeros((pos.shape[0], C), _F32)
    for li, f_ref in ((2, f2_ref), (3, f3_ref)):
        hw = LVL_HW[li][0] * LVL_HW[li][1]
        iot = lax.broadcasted_iota(jnp.int32, (1, hw), 1).astype(_F32)
        ohs = []
        for v in range(NV):
            q = li * 6 + v
            oh = (cw00[:, q:q + 1] * (pix00[:, q:q + 1] == iot).astype(_F32)
                  + cw01[:, q:q + 1] * (pix01[:, q:q + 1] == iot).astype(_F32)
                  + cw10[:, q:q + 1] * (pix10[:, q:q + 1] == iot).astype(_F32)
                  + cw11[:, q:q + 1] * (pix11[:, q:q + 1] == iot).astype(_F32))
            ohs.append(oh)
        ohcat = jnp.concatenate(ohs, axis=1)
        s23 = s23 + lax.dot_general(ohcat, f_ref[...],
                                    (((1,), (0,)), ((), ())), precision=None)
    pe_ref[...] = pe + s23


def _run_precomp(pos_p, a0, a1, a2, pe_w1, pe_b1, pe_w2, pe_b2, wgt_w, wgt_b,
                 f2t, f3t):
    n_blk = QPP // BLK_A
    full = lambda shape: pl.BlockSpec(shape, lambda i: (0,) * len(shape))
    return pl.pallas_call(
        _precomp_body,
        grid=(n_blk,),
        in_specs=[
            pl.BlockSpec((BLK_A, 3), lambda i: (i, 0)),
            full((4, 24)), full((4, 24)), full((4, 24)),
            full((48, 256)), full((1, 256)),
            full((256, 128)), full((1, 128)),
            full((128, 4)), full((1, 4)),
            full((3, 48)), full((1, 48)), full((8, 24)),
            full((6 * 8 * 22, C)), full((6 * 4 * 11, C)),
        ],
        out_specs=[
            pl.BlockSpec((BLK_A, NJR), lambda i: (i, 0)),
            pl.BlockSpec((BLK_A, NJ), lambda i: (i, 0)),
            pl.BlockSpec((BLK_A, C), lambda i: (i, 0)),
        ],
        out_shape=[
            jax.ShapeDtypeStruct((QPP, NJR), jnp.int32),
            jax.ShapeDtypeStruct((QPP, NJ), _F32),
            jax.ShapeDtypeStruct((QPP, C), _F32),
        ],
    )(pos_p, a0, a1, a2, pe_w1, pe_b1, pe_w2, pe_b2, wgt_w, wgt_b,
      jnp.asarray(_P48), jnp.asarray(_PH48), jnp.asarray(_LC), f2t, f3t)


def _bcast_lane(vec, t):
    dn = lax.GatherDimensionNumbers(
        offset_dims=(), collapsed_slice_dims=(0,), start_index_map=(0,))
    return lax.gather(vec, jnp.full((16, 1), t, jnp.int32), dn, (1,),
                      mode=lax.GatherScatterMode.PROMISE_IN_BOUNDS)


def _sc_body(idx_hbm, wgt_hbm, pe_hbm, table_hbm, out_hbm,
             idx_v, wgt_v, acc_v, rows_v, *sems):
    cc = lax.axis_index("c")
    ss = lax.axis_index("s")
    wid = ss * 2 + cc
    base = wid * PTS_W
    hi_mask = jnp.full((16,), np.int32(-65536), jnp.int32)

    def mac_point(p, b, pp):
        accs = [acc_v[p, pl.ds(cch * 16, 16)] for cch in range(8)]
        wvecs = [wgt_v[p, pl.ds(k * 16, 16)] for k in range(NJ // 16)]
        for j in range(NJR):
            ws = []
            for cslot in range(4):
                lw = cslot * 24 + j
                ws.append(_bcast_lane(wvecs[lw // 16], lw % 16))
            flat = pp * NJR + j
            d = flat // DIDX
            s = flat - d * DIDX
            for c4 in range(4):
                lo = accs[2 * c4]
                hi = accs[2 * c4 + 1]
                for cslot in range(4):
                    pw = rows_v[b, d, s, pl.ds(cslot * 64 + c4 * 16, 16)]
                    ea = lax.bitcast_convert_type(pw << 16, _F32)
                    eb = lax.bitcast_convert_type(pw & hi_mask, _F32)
                    lo = lo + ws[cslot] * ea
                    hi = hi + ws[cslot] * eb
                accs[2 * c4] = lo
                accs[2 * c4 + 1] = hi
        for cch in range(8):
            acc_v[p, pl.ds(cch * 16, 16)] = accs[cch]

    def fire(g, b):
        for t in range(ND):
            pltpu.async_copy(table_hbm.at[idx_v.at[g * ND + t, 0]],
                             rows_v.at[b, t], sems[b])

    def drain(g, b):
        for t in range(ND):
            pltpu.make_async_copy(table_hbm.at[idx_v.at[g * ND + t, 0]],
                                  rows_v.at[b, t], sems[b]).wait()

    def mac_group(g, b):
        def body(pp, carry):
            mac_point(g * KG + pp, b, pp)
            return carry

        lax.fori_loop(0, KG, body, 0)

    IR = CH * NJR // DIDX

    def chunk_body(ci, carry):
        cb = base + ci * CH
        pltpu.sync_copy(idx_hbm.at[pl.ds((wid * PTS_W // CH + ci) * IR, IR)],
                        idx_v)
        pltpu.sync_copy(wgt_hbm.at[pl.ds(cb, CH)], wgt_v)
        pltpu.sync_copy(pe_hbm.at[pl.ds(cb, CH)], acc_v)
        fire(0, 0)
        fire(1, 1)

        def gpair(i, carry2):
            g0 = 2 * i
            drain(g0, 0)
            mac_group(g0, 0)

            @pl.when(g0 + 2 < NGC)
            def _():
                fire(g0 + 2, 0)

            drain(g0 + 1, 1)
            mac_group(g0 + 1, 1)

            @pl.when(g0 + 3 < NGC)
            def _():
                fire(g0 + 3, 1)

            return carry2

        lax.fori_loop(0, NGC // 2, gpair, 0)
        pltpu.sync_copy(acc_v, out_hbm.at[pl.ds(cb, CH)])
        return carry

    lax.fori_loop(0, PTS_W // CH, chunk_body, 0)


def _run_sc_gather(idx, wgt, pe, table):
    mesh = plsc.VectorSubcoreMesh(core_axis_name="c", subcore_axis_name="s")
    fn = functools.partial(
        pl.kernel,
        mesh=mesh,
        out_type=jax.ShapeDtypeStruct((QPP, C), _F32),
        scratch_types=[
            pltpu.VMEM((CH * NJR // DIDX, 1, DIDX), jnp.int32),
            pltpu.VMEM((CH, NJ), _F32),
            pltpu.VMEM((CH, C), _F32),
            pltpu.VMEM((2, ND, DIDX, 2 * C), jnp.int32),
        ] + [pltpu.SemaphoreType.DMA] * 2,
    )(_sc_body)
    return fn(idx, wgt, pe, table)


def _hmlp_body(x_ref, w1_ref, b1_ref, w2_ref, b2_ref, w3_ref, b3_ref,
               w4_ref, b4_ref, o_ref):
    x = x_ref[...]
    h = jnp.maximum(
        lax.dot_general(x, w1_ref[...], (((1,), (0,)), ((), ())),
                        precision=None) + b1_ref[...], 0.0)
    h = jnp.maximum(
        lax.dot_general(h, w2_ref[...], (((1,), (0,)), ((), ())),
                        precision=None) + b2_ref[...], 0.0)
    h = jnp.maximum(
        lax.dot_general(h, w3_ref[...], (((1,), (0,)), ((), ())),
                        precision=None) + b3_ref[...], 0.0)
    o_ref[...] = lax.dot_general(h, w4_ref[...], (((1,), (0,)), ((), ())),
                                 precision=None) + b4_ref[...]


NQP = 2560
BLK_C = 512


def _run_hmlp(x, w1, b1, w2, b2, w3, b3, w4, b4):
    full = lambda shape: pl.BlockSpec(shape, lambda i: (0,) * len(shape))
    return pl.pallas_call(
        _hmlp_body,
        grid=(NQP // BLK_C,),
        in_specs=[
            pl.BlockSpec((BLK_C, NZ * C), lambda i: (i, 0)),
            full((NZ * C, 1024)), full((1, 1024)),
            full((1024, 1024)), full((1, 1024)),
            full((1024, 1024)), full((1, 1024)),
            full((1024, C)), full((1, C)),
        ],
        out_specs=pl.BlockSpec((BLK_C, C), lambda i: (i, 0)),
        out_shape=jax.ShapeDtypeStruct((NQP, C), _F32),
    )(x, w1, b1, w2, b2, w3, b3, w4, b4)


def _flatten_feats(feat0, feat1, feat2, feat3):
    rows = []
    for f in (feat0, feat1):
        g = jnp.transpose(f[0], (0, 2, 3, 1))
        v, h, w, c = g.shape
        pr = jnp.concatenate(
            [g[:, :-1, :-1, :], g[:, :-1, 1:, :],
             g[:, 1:, :-1, :], g[:, 1:, 1:, :]], axis=-1)
        rows.append(pr.reshape(v * (h - 1) * (w - 1), 4 * c))
    t = jnp.concatenate(rows, axis=0)
    t32 = t.reshape(N_ROWS, 16, 32)
    t = jnp.stack([t32[..., :16], t32[..., 16:]], axis=-1).reshape(N_ROWS, 4 * C)
    t16 = t.astype(jnp.bfloat16).reshape(N_ROWS, 2 * C, 2)
    return jax.lax.bitcast_convert_type(t16, jnp.int32)


def kernel(feat0, feat1, feat2, feat3, reference_points, lidar2img,
           pe_w1, pe_b1, pe_w2, pe_b2, wgt_w, wgt_b,
           hm_w1, hm_b1, hm_w2, hm_b2, hm_w3, hm_b3, hm_w4, hm_b4):
    pos = jnp.transpose(reference_points[0], (1, 0, 2)).reshape(QP, 3)
    pos_p = jnp.pad(pos, ((0, QPP - QP), (0, 0)))
    m = lidar2img[0, 0]
    a0 = jnp.tile(m[:, 0, :], (NL, 1)).T
    a1 = jnp.tile(m[:, 1, :], (NL, 1)).T
    a2 = jnp.tile(m[:, 2, :], (NL, 1)).T
    table = _flatten_feats(feat0, feat1, feat2, feat3)
    f2t = jnp.transpose(feat2[0], (0, 2, 3, 1)).reshape(6 * 8 * 22, C)
    f3t = jnp.transpose(feat3[0], (0, 2, 3, 1)).reshape(6 * 4 * 11, C)

    idx, wgt, pe = _run_precomp(
        pos_p, a0, a1, a2,
        pe_w1, pe_b1.reshape(1, -1), pe_w2, pe_b2.reshape(1, -1),
        wgt_w, wgt_b.reshape(1, -1), f2t, f3t)

    idx = idx.reshape(QPP * NJR // DIDX, 1, DIDX)
    sampled = _run_sc_gather(idx, wgt, pe, table)

    x = sampled[:QP].reshape(NQ, NZ * C)
    x = jnp.pad(x, ((0, NQP - NQ), (0, 0)))
    out = _run_hmlp(
        x, hm_w1, hm_b1.reshape(1, -1), hm_w2, hm_b2.reshape(1, -1),
        hm_w3, hm_b3.reshape(1, -1), hm_w4, hm_b4.reshape(1, -1))
    return out[:NQ].reshape(1, NQ, C)


# speedup vs baseline: 1.1375x; 1.0822x over previous
import functools

import numpy as np
import jax
import jax.numpy as jnp
from jax import lax
from jax.experimental import pallas as pl
from jax.experimental.pallas import tpu as pltpu
from jax.experimental.pallas import tpu_sc as plsc

PC_RANGE = [-51.2, -51.2, -5.0, 51.2, 51.2, 3.0]
IMG_H, IMG_W = 256, 704
LVL_HW = ((32, 88), (16, 44), (8, 22), (4, 11))
NV = 6
NL = 4
NF = 8
C = 128
NQ = 2500
NZ = 4
QP = NQ * NZ
QPP = 10240
NJ = NV * NL * 4
NJR = NV * NL
DIDX = 64

BLK_A = 512

NW = 32
PTS_W = QPP // NW
CH = 32
KG = 8
ND = KG * NJR // DIDX
NGC = CH // KG

_LVL_BASE = []
_acc = 0
for _h, _w in LVL_HW:
    _LVL_BASE.append(_acc)
    _acc += NV * (_h - 1) * (_w - 1)
N_ROWS = _acc

_F32 = jnp.float32
_HI = lax.Precision.HIGHEST


def _np_lane_consts():
    wscale = np.zeros((1, 24), np.float32)
    hscale = np.zeros((1, 24), np.float32)
    wm1 = np.zeros((1, 24), np.float32)
    hm1 = np.zeros((1, 24), np.float32)
    wl = np.zeros((1, 24), np.float32)
    base = np.zeros((1, 24), np.float32)
    for v in range(NV):
        for l in range(NL):
            h, w = LVL_HW[l]
            j = v * 4 + l
            wscale[0, j] = w / IMG_W
            hscale[0, j] = h / IMG_H
            wm1[0, j] = w - 1
            hm1[0, j] = h - 1
            wl[0, j] = w - 1
            base[0, j] = _LVL_BASE[l] + v * (h - 1) * (w - 1)
    return wscale, hscale, wm1, hm1, wl, base


_WSCALE, _HSCALE, _WM1, _HM1, _WL, _BASE = _np_lane_consts()


def _np_pe_consts():
    p48 = np.zeros((3, 48), np.float32)
    ph48 = np.zeros((1, 48), np.float32)
    for d in range(3):
        for s in range(16):
            col = d * 16 + s
            f = s % 8
            p48[d, col] = (2.0 ** f) * np.pi
            ph48[0, col] = 0.0 if s < 8 else np.pi / 2.0
    return p48, ph48


_P48, _PH48 = _np_pe_consts()
_SCALE3 = np.array([[PC_RANGE[3] - PC_RANGE[0],
                     PC_RANGE[4] - PC_RANGE[1],
                     PC_RANGE[5] - PC_RANGE[2]]], np.float32)
_OFF3 = np.array([[PC_RANGE[0], PC_RANGE[1], PC_RANGE[2]]], np.float32)

_LC = np.zeros((8, 24), np.float32)
_LC[0:1] = _WSCALE
_LC[1:2] = _HSCALE
_LC[2:3] = _WM1
_LC[3:4] = _HM1
_LC[4:5] = _WL
_LC[5:6] = _BASE
_LC[6, 0:3] = _SCALE3[0]
_LC[7, 0:3] = _OFF3[0]


def _precomp_body(pos_ref, a0_ref, a1_ref, a2_ref,
                  pw1_ref, pb1_ref, pw2_ref, pb2_ref, ww_ref, wb_ref,
                  p48_ref, ph48_ref, lc_ref,
                  idx_ref, wgt_ref, pe_ref):
    pos = pos_ref[...]
    ang = lax.dot_general(pos, p48_ref[...],
                          (((1,), (0,)), ((), ())), precision=_HI)
    enc = jnp.sin(ang + ph48_ref[...])
    h = jnp.maximum(
        lax.dot_general(enc, pw1_ref[...], (((1,), (0,)), ((), ())),
                        precision=None) + pb1_ref[...], 0.0)
    pe = lax.dot_general(h, pw2_ref[...], (((1,), (0,)), ((), ())),
                         precision=None) + pb2_ref[...]
    pe_ref[...] = pe
    logits = lax.dot_general(pe, ww_ref[...], (((1,), (0,)), ((), ())),
                             precision=None) + wb_ref[...]
    m = jnp.max(logits, axis=1, keepdims=True)
    e = jnp.exp(logits - m)
    sw = e / jnp.sum(e, axis=1, keepdims=True)
    sw24 = jnp.concatenate([sw] * NV, axis=1)
    lc = lc_ref[...]
    rp = pos * lc[6:7, 0:3] + lc[7:8, 0:3]
    rph = jnp.concatenate(
        [rp, jnp.ones((rp.shape[0], 1), _F32)], axis=1)
    cx = lax.dot_general(rph, a0_ref[...], (((1,), (0,)), ((), ())),
                         precision=None)
    cy = lax.dot_general(rph, a1_ref[...], (((1,), (0,)), ((), ())),
                         precision=None)
    cz = lax.dot_general(rph, a2_ref[...], (((1,), (0,)), ((), ())),
                         precision=None)
    homo = jnp.maximum(cz, 1e-5)
    xs = cx / homo * lc[0:1, :] - 0.5
    ys = cy / homo * lc[1:2, :] - 0.5
    x0 = jnp.floor(xs)
    y0 = jnp.floor(ys)
    fx = xs - x0
    fy = ys - y0
    wm1 = lc[2:3, :]
    hm1 = lc[3:4, :]
    vx0 = ((x0 >= 0.0) & (x0 <= wm1)).astype(_F32)
    vx1 = ((x0 + 1.0 >= 0.0) & (x0 + 1.0 <= wm1)).astype(_F32)
    vy0 = ((y0 >= 0.0) & (y0 <= hm1)).astype(_F32)
    vy1 = ((y0 + 1.0 >= 0.0) & (y0 + 1.0 <= hm1)).astype(_F32)
    xc0 = jnp.clip(x0, 0.0, wm1)
    xc1 = jnp.clip(x0 + 1.0, 0.0, wm1)
    yc0 = jnp.clip(y0, 0.0, hm1)
    yc1 = jnp.clip(y0 + 1.0, 0.0, hm1)
    xl = jnp.clip(x0, 0.0, wm1 - 1.0)
    yl = jnp.clip(y0, 0.0, hm1 - 1.0)
    wx0 = (1.0 - fx) * vx0
    wx1 = fx * vx1
    wy0 = (1.0 - fy) * vy0
    wy1 = fy * vy1
    whx0 = (wx0 * (xc0 == xl).astype(_F32)
            + wx1 * (xc1 == xl).astype(_F32))
    whx1 = (wx0 * (xc0 == xl + 1.0).astype(_F32)
            + wx1 * (xc1 == xl + 1.0).astype(_F32))
    why0 = (wy0 * (yc0 == yl).astype(_F32)
            + wy1 * (yc1 == yl).astype(_F32))
    why1 = (wy0 * (yc0 == yl + 1.0).astype(_F32)
            + wy1 * (yc1 == yl + 1.0).astype(_F32))
    stride = lc[4:5, :]
    pbase = lc[5:6, :]
    idx_ref[...] = (pbase + yl * stride + xl).astype(jnp.int32)
    wgt_ref[...] = jnp.concatenate(
        [why0 * whx0 * sw24, why0 * whx1 * sw24,
         why1 * whx0 * sw24, why1 * whx1 * sw24], axis=1)


def _run_precomp(pos_p, a0, a1, a2, pe_w1, pe_b1, pe_w2, pe_b2, wgt_w, wgt_b):
    n_blk = QPP // BLK_A
    full = lambda shape: pl.BlockSpec(shape, lambda i: (0,) * len(shape))
    return pl.pallas_call(
        _precomp_body,
        grid=(n_blk,),
        in_specs=[
            pl.BlockSpec((BLK_A, 3), lambda i: (i, 0)),
            full((4, 24)), full((4, 24)), full((4, 24)),
            full((48, 256)), full((1, 256)),
            full((256, 128)), full((1, 128)),
            full((128, 4)), full((1, 4)),
            full((3, 48)), full((1, 48)), full((8, 24)),
        ],
        out_specs=[
            pl.BlockSpec((BLK_A, NJR), lambda i: (i, 0)),
            pl.BlockSpec((BLK_A, NJ), lambda i: (i, 0)),
            pl.BlockSpec((BLK_A, C), lambda i: (i, 0)),
        ],
        out_shape=[
            jax.ShapeDtypeStruct((QPP, NJR), jnp.int32),
            jax.ShapeDtypeStruct((QPP, NJ), _F32),
            jax.ShapeDtypeStruct((QPP, C), _F32),
        ],
    )(pos_p, a0, a1, a2, pe_w1, pe_b1, pe_w2, pe_b2, wgt_w, wgt_b,
      jnp.asarray(_P48), jnp.asarray(_PH48), jnp.asarray(_LC))


def _bcast_lane(vec, t):
    dn = lax.GatherDimensionNumbers(
        offset_dims=(), collapsed_slice_dims=(0,), start_index_map=(0,))
    return lax.gather(vec, jnp.full((16, 1), t, jnp.int32), dn, (1,),
                      mode=lax.GatherScatterMode.PROMISE_IN_BOUNDS)


def _sc_body(idx_hbm, wgt_hbm, pe_hbm, table_hbm, out_hbm,
             idx_v, wgt_v, acc_v, rows_v, *sems):
    cc = lax.axis_index("c")
    ss = lax.axis_index("s")
    wid = ss * 2 + cc
    base = wid * PTS_W
    hi_mask = jnp.full((16,), np.int32(-65536), jnp.int32)

    def mac_point(p, b, pp):
        accs = [acc_v[p, pl.ds(cch * 16, 16)] for cch in range(8)]
        wvecs = [wgt_v[p, pl.ds(k * 16, 16)] for k in range(NJ // 16)]
        for j in range(NJR):
            ws = []
            for cslot in range(4):
                lw = cslot * 24 + j
                ws.append(_bcast_lane(wvecs[lw // 16], lw % 16))
            flat = pp * NJR + j
            d = flat // DIDX
            s = flat - d * DIDX
            for c4 in range(4):
                lo = accs[2 * c4]
                hi = accs[2 * c4 + 1]
                for cslot in range(4):
                    pw = rows_v[b, d, s, pl.ds(cslot * 64 + c4 * 16, 16)]
                    ea = lax.bitcast_convert_type(pw << 16, _F32)
                    eb = lax.bitcast_convert_type(pw & hi_mask, _F32)
                    lo = lo + ws[cslot] * ea
                    hi = hi + ws[cslot] * eb
                accs[2 * c4] = lo
                accs[2 * c4 + 1] = hi
        for cch in range(8):
            acc_v[p, pl.ds(cch * 16, 16)] = accs[cch]

    def fire(g, b):
        for t in range(ND):
            pltpu.async_copy(table_hbm.at[idx_v.at[g * ND + t, 0]],
                             rows_v.at[b, t], sems[b])

    def drain(g, b):
        for t in range(ND):
            pltpu.make_async_copy(table_hbm.at[idx_v.at[g * ND + t, 0]],
                                  rows_v.at[b, t], sems[b]).wait()

    def mac_group(g, b):
        def body(pp, carry):
            mac_point(g * KG + pp, b, pp)
            return carry

        lax.fori_loop(0, KG, body, 0)

    IR = CH * NJR // DIDX

    def chunk_body(ci, carry):
        cb = base + ci * CH
        pltpu.sync_copy(idx_hbm.at[pl.ds((wid * PTS_W // CH + ci) * IR, IR)],
                        idx_v)
        pltpu.sync_copy(wgt_hbm.at[pl.ds(cb, CH)], wgt_v)
        pltpu.sync_copy(pe_hbm.at[pl.ds(cb, CH)], acc_v)
        fire(0, 0)
        fire(1, 1)

        def gpair(i, carry2):
            g0 = 2 * i
            drain(g0, 0)
            mac_group(g0, 0)

            @pl.when(g0 + 2 < NGC)
            def _():
                fire(g0 + 2, 0)

            drain(g0 + 1, 1)
            mac_group(g0 + 1, 1)

            @pl.when(g0 + 3 < NGC)
            def _():
                fire(g0 + 3, 1)

            return carry2

        lax.fori_loop(0, NGC // 2, gpair, 0)
        pltpu.sync_copy(acc_v, out_hbm.at[pl.ds(cb, CH)])
        return carry

    lax.fori_loop(0, PTS_W // CH, chunk_body, 0)


def _run_sc_gather(idx, wgt, pe, table):
    mesh = plsc.VectorSubcoreMesh(core_axis_name="c", subcore_axis_name="s")
    fn = functools.partial(
        pl.kernel,
        mesh=mesh,
        out_type=jax.ShapeDtypeStruct((QPP, C), _F32),
        scratch_types=[
            pltpu.VMEM((CH * NJR // DIDX, 1, DIDX), jnp.int32),
            pltpu.VMEM((CH, NJ), _F32),
            pltpu.VMEM((CH, C), _F32),
            pltpu.VMEM((2, ND, DIDX, 2 * C), jnp.int32),
        ] + [pltpu.SemaphoreType.DMA] * 2,
    )(_sc_body)
    return fn(idx, wgt, pe, table)


def _hmlp_body(x_ref, w1_ref, b1_ref, w2_ref, b2_ref, w3_ref, b3_ref,
               w4_ref, b4_ref, o_ref):
    x = x_ref[...]
    h = jnp.maximum(
        lax.dot_general(x, w1_ref[...], (((1,), (0,)), ((), ())),
                        precision=None) + b1_ref[...], 0.0)
    h = jnp.maximum(
        lax.dot_general(h, w2_ref[...], (((1,), (0,)), ((), ())),
                        precision=None) + b2_ref[...], 0.0)
    h = jnp.maximum(
        lax.dot_general(h, w3_ref[...], (((1,), (0,)), ((), ())),
                        precision=None) + b3_ref[...], 0.0)
    o_ref[...] = lax.dot_general(h, w4_ref[...], (((1,), (0,)), ((), ())),
                                 precision=None) + b4_ref[...]


NQP = 2560
BLK_C = 512


def _run_hmlp(x, w1, b1, w2, b2, w3, b3, w4, b4):
    full = lambda shape: pl.BlockSpec(shape, lambda i: (0,) * len(shape))
    return pl.pallas_call(
        _hmlp_body,
        grid=(NQP // BLK_C,),
        in_specs=[
            pl.BlockSpec((BLK_C, NZ * C), lambda i: (i, 0)),
            full((NZ * C, 1024)), full((1, 1024)),
            full((1024, 1024)), full((1, 1024)),
            full((1024, 1024)), full((1, 1024)),
            full((1024, C)), full((1, C)),
        ],
        out_specs=pl.BlockSpec((BLK_C, C), lambda i: (i, 0)),
        out_shape=jax.ShapeDtypeStruct((NQP, C), _F32),
    )(x, w1, b1, w2, b2, w3, b3, w4, b4)


def _flatten_feats(feat0, feat1, feat2, feat3):
    rows = []
    for f in (feat0, feat1, feat2, feat3):
        g = jnp.transpose(f[0], (0, 2, 3, 1))
        v, h, w, c = g.shape
        pr = jnp.concatenate(
            [g[:, :-1, :-1, :], g[:, :-1, 1:, :],
             g[:, 1:, :-1, :], g[:, 1:, 1:, :]], axis=-1)
        rows.append(pr.reshape(v * (h - 1) * (w - 1), 4 * c))
    t = jnp.concatenate(rows, axis=0)
    t32 = t.reshape(N_ROWS, 16, 32)
    t = jnp.stack([t32[..., :16], t32[..., 16:]], axis=-1).reshape(N_ROWS, 4 * C)
    t16 = t.astype(jnp.bfloat16).reshape(N_ROWS, 2 * C, 2)
    return jax.lax.bitcast_convert_type(t16, jnp.int32)


def kernel(feat0, feat1, feat2, feat3, reference_points, lidar2img,
           pe_w1, pe_b1, pe_w2, pe_b2, wgt_w, wgt_b,
           hm_w1, hm_b1, hm_w2, hm_b2, hm_w3, hm_b3, hm_w4, hm_b4):
    pos = jnp.transpose(reference_points[0], (1, 0, 2)).reshape(QP, 3)
    pos_p = jnp.pad(pos, ((0, QPP - QP), (0, 0)))
    m = lidar2img[0, 0]
    a0 = jnp.repeat(m[:, 0, :], NL, axis=0).T
    a1 = jnp.repeat(m[:, 1, :], NL, axis=0).T
    a2 = jnp.repeat(m[:, 2, :], NL, axis=0).T
    table = _flatten_feats(feat0, feat1, feat2, feat3)

    idx, wgt, pe = _run_precomp(
        pos_p, a0, a1, a2,
        pe_w1, pe_b1.reshape(1, -1), pe_w2, pe_b2.reshape(1, -1),
        wgt_w, wgt_b.reshape(1, -1))

    idx = idx.reshape(QPP * NJR // DIDX, 1, DIDX)
    sampled = _run_sc_gather(idx, wgt, pe, table)

    x = sampled[:QP].reshape(NQ, NZ * C)
    x = jnp.pad(x, ((0, NQP - NQ), (0, 0)))
    out = _run_hmlp(
        x, hm_w1, hm_b1.reshape(1, -1), hm_w2, hm_b2.reshape(1, -1),
        hm_w3, hm_b3.reshape(1, -1), hm_w4, hm_b4.reshape(1, -1))
    return out[:NQ].reshape(1, NQ, C)


# speedup vs baseline: 1.1462x; 1.0076x over previous
import functools

import numpy as np
import jax
import jax.numpy as jnp
from jax import lax
from jax.experimental import pallas as pl
from jax.experimental.pallas import tpu as pltpu
from jax.experimental.pallas import tpu_sc as plsc

PC_RANGE = [-51.2, -51.2, -5.0, 51.2, 51.2, 3.0]
IMG_H, IMG_W = 256, 704
LVL_HW = ((32, 88), (16, 44), (8, 22), (4, 11))
NV = 6
NL = 4
NF = 8
C = 128
NQ = 2500
NZ = 4
QP = NQ * NZ
QPP = 10240
NJ = NV * NL * 4
NJR = NV * NL
DIDX = 64

BLK_A = 512

NW = 32
PTS_W = QPP // NW
CH = 64
KG = 8
ND = KG * NJR // DIDX
NGC = CH // KG

_LVL_BASE = []
_acc = 0
for _h, _w in LVL_HW:
    _LVL_BASE.append(_acc)
    _acc += NV * (_h - 1) * (_w - 1)
N_ROWS = _acc

_F32 = jnp.float32
_HI = lax.Precision.HIGHEST


def _np_lane_consts():
    wscale = np.zeros((1, 24), np.float32)
    hscale = np.zeros((1, 24), np.float32)
    wm1 = np.zeros((1, 24), np.float32)
    hm1 = np.zeros((1, 24), np.float32)
    wl = np.zeros((1, 24), np.float32)
    base = np.zeros((1, 24), np.float32)
    for v in range(NV):
        for l in range(NL):
            h, w = LVL_HW[l]
            j = v * 4 + l
            wscale[0, j] = w / IMG_W
            hscale[0, j] = h / IMG_H
            wm1[0, j] = w - 1
            hm1[0, j] = h - 1
            wl[0, j] = w - 1
            base[0, j] = _LVL_BASE[l] + v * (h - 1) * (w - 1)
    return wscale, hscale, wm1, hm1, wl, base


_WSCALE, _HSCALE, _WM1, _HM1, _WL, _BASE = _np_lane_consts()


def _np_pe_consts():
    p48 = np.zeros((3, 48), np.float32)
    ph48 = np.zeros((1, 48), np.float32)
    for d in range(3):
        for s in range(16):
            col = d * 16 + s
            f = s % 8
            p48[d, col] = (2.0 ** f) * np.pi
            ph48[0, col] = 0.0 if s < 8 else np.pi / 2.0
    return p48, ph48


_P48, _PH48 = _np_pe_consts()
_SCALE3 = np.array([[PC_RANGE[3] - PC_RANGE[0],
                     PC_RANGE[4] - PC_RANGE[1],
                     PC_RANGE[5] - PC_RANGE[2]]], np.float32)
_OFF3 = np.array([[PC_RANGE[0], PC_RANGE[1], PC_RANGE[2]]], np.float32)

_LC = np.zeros((8, 24), np.float32)
_LC[0:1] = _WSCALE
_LC[1:2] = _HSCALE
_LC[2:3] = _WM1
_LC[3:4] = _HM1
_LC[4:5] = _WL
_LC[5:6] = _BASE
_LC[6, 0:3] = _SCALE3[0]
_LC[7, 0:3] = _OFF3[0]


def _precomp_body(pos_ref, a0_ref, a1_ref, a2_ref,
                  pw1_ref, pb1_ref, pw2_ref, pb2_ref, ww_ref, wb_ref,
                  p48_ref, ph48_ref, lc_ref,
                  idx_ref, wgt_ref, pe_ref):
    pos = pos_ref[...]
    ang = lax.dot_general(pos, p48_ref[...],
                          (((1,), (0,)), ((), ())), precision=_HI)
    enc = jnp.sin(ang + ph48_ref[...])
    h = jnp.maximum(
        lax.dot_general(enc, pw1_ref[...], (((1,), (0,)), ((), ())),
                        precision=None) + pb1_ref[...], 0.0)
    pe = lax.dot_general(h, pw2_ref[...], (((1,), (0,)), ((), ())),
                         precision=None) + pb2_ref[...]
    pe_ref[...] = pe
    logits = lax.dot_general(pe, ww_ref[...], (((1,), (0,)), ((), ())),
                             precision=None) + wb_ref[...]
    m = jnp.max(logits, axis=1, keepdims=True)
    e = jnp.exp(logits - m)
    sw = e / jnp.sum(e, axis=1, keepdims=True)
    sw24 = jnp.concatenate([sw] * NV, axis=1)
    lc = lc_ref[...]
    rp = pos * lc[6:7, 0:3] + lc[7:8, 0:3]
    rph = jnp.concatenate(
        [rp, jnp.ones((rp.shape[0], 1), _F32)], axis=1)
    cx = lax.dot_general(rph, a0_ref[...], (((1,), (0,)), ((), ())),
                         precision=None)
    cy = lax.dot_general(rph, a1_ref[...], (((1,), (0,)), ((), ())),
                         precision=None)
    cz = lax.dot_general(rph, a2_ref[...], (((1,), (0,)), ((), ())),
                         precision=None)
    homo = jnp.maximum(cz, 1e-5)
    xs = cx / homo * lc[0:1, :] - 0.5
    ys = cy / homo * lc[1:2, :] - 0.5
    x0 = jnp.floor(xs)
    y0 = jnp.floor(ys)
    fx = xs - x0
    fy = ys - y0
    wm1 = lc[2:3, :]
    hm1 = lc[3:4, :]
    vx0 = ((x0 >= 0.0) & (x0 <= wm1)).astype(_F32)
    vx1 = ((x0 + 1.0 >= 0.0) & (x0 + 1.0 <= wm1)).astype(_F32)
    vy0 = ((y0 >= 0.0) & (y0 <= hm1)).astype(_F32)
    vy1 = ((y0 + 1.0 >= 0.0) & (y0 + 1.0 <= hm1)).astype(_F32)
    xc0 = jnp.clip(x0, 0.0, wm1)
    xc1 = jnp.clip(x0 + 1.0, 0.0, wm1)
    yc0 = jnp.clip(y0, 0.0, hm1)
    yc1 = jnp.clip(y0 + 1.0, 0.0, hm1)
    xl = jnp.clip(x0, 0.0, wm1 - 1.0)
    yl = jnp.clip(y0, 0.0, hm1 - 1.0)
    wx0 = (1.0 - fx) * vx0
    wx1 = fx * vx1
    wy0 = (1.0 - fy) * vy0
    wy1 = fy * vy1
    whx0 = (wx0 * (xc0 == xl).astype(_F32)
            + wx1 * (xc1 == xl).astype(_F32))
    whx1 = (wx0 * (xc0 == xl + 1.0).astype(_F32)
            + wx1 * (xc1 == xl + 1.0).astype(_F32))
    why0 = (wy0 * (yc0 == yl).astype(_F32)
            + wy1 * (yc1 == yl).astype(_F32))
    why1 = (wy0 * (yc0 == yl + 1.0).astype(_F32)
            + wy1 * (yc1 == yl + 1.0).astype(_F32))
    stride = lc[4:5, :]
    pbase = lc[5:6, :]
    idx_ref[...] = (pbase + yl * stride + xl).astype(jnp.int32)
    wgt_ref[...] = jnp.concatenate(
        [why0 * whx0 * sw24, why0 * whx1 * sw24,
         why1 * whx0 * sw24, why1 * whx1 * sw24], axis=1)


def _run_precomp(pos_p, a0, a1, a2, pe_w1, pe_b1, pe_w2, pe_b2, wgt_w, wgt_b):
    n_blk = QPP // BLK_A
    full = lambda shape: pl.BlockSpec(shape, lambda i: (0,) * len(shape))
    return pl.pallas_call(
        _precomp_body,
        grid=(n_blk,),
        in_specs=[
            pl.BlockSpec((BLK_A, 3), lambda i: (i, 0)),
            full((4, 24)), full((4, 24)), full((4, 24)),
            full((48, 256)), full((1, 256)),
            full((256, 128)), full((1, 128)),
            full((128, 4)), full((1, 4)),
            full((3, 48)), full((1, 48)), full((8, 24)),
        ],
        out_specs=[
            pl.BlockSpec((BLK_A, NJR), lambda i: (i, 0)),
            pl.BlockSpec((BLK_A, NJ), lambda i: (i, 0)),
            pl.BlockSpec((BLK_A, C), lambda i: (i, 0)),
        ],
        out_shape=[
            jax.ShapeDtypeStruct((QPP, NJR), jnp.int32),
            jax.ShapeDtypeStruct((QPP, NJ), _F32),
            jax.ShapeDtypeStruct((QPP, C), _F32),
        ],
    )(pos_p, a0, a1, a2, pe_w1, pe_b1, pe_w2, pe_b2, wgt_w, wgt_b,
      jnp.asarray(_P48), jnp.asarray(_PH48), jnp.asarray(_LC))


def _bcast_lane(vec, t):
    dn = lax.GatherDimensionNumbers(
        offset_dims=(), collapsed_slice_dims=(0,), start_index_map=(0,))
    return lax.gather(vec, jnp.full((16, 1), t, jnp.int32), dn, (1,),
                      mode=lax.GatherScatterMode.PROMISE_IN_BOUNDS)


def _sc_body(idx_hbm, wgt_hbm, pe_hbm, table_hbm, out_hbm,
             idx_v, wgt_v, acc_v, rows_v, *sems):
    cc = lax.axis_index("c")
    ss = lax.axis_index("s")
    wid = ss * 2 + cc
    base = wid * PTS_W
    hi_mask = jnp.full((16,), np.int32(-65536), jnp.int32)

    def mac_point(p, b, pp):
        accs = [acc_v[p, pl.ds(cch * 16, 16)] for cch in range(8)]
        wvecs = [wgt_v[p, pl.ds(k * 16, 16)] for k in range(NJ // 16)]
        for j in range(NJR):
            ws = []
            for cslot in range(4):
                lw = cslot * 24 + j
                ws.append(_bcast_lane(wvecs[lw // 16], lw % 16))
            flat = pp * NJR + j
            d = flat // DIDX
            s = flat - d * DIDX
            for c4 in range(4):
                lo = accs[2 * c4]
                hi = accs[2 * c4 + 1]
                for cslot in range(4):
                    pw = rows_v[b, d, s, pl.ds(cslot * 64 + c4 * 16, 16)]
                    ea = lax.bitcast_convert_type(pw << 16, _F32)
                    eb = lax.bitcast_convert_type(pw & hi_mask, _F32)
                    lo = lo + ws[cslot] * ea
                    hi = hi + ws[cslot] * eb
                accs[2 * c4] = lo
                accs[2 * c4 + 1] = hi
        for cch in range(8):
            acc_v[p, pl.ds(cch * 16, 16)] = accs[cch]

    def fire(g, b):
        for t in range(ND):
            pltpu.async_copy(table_hbm.at[idx_v.at[g * ND + t, 0]],
                             rows_v.at[b, t], sems[b])

    def drain(g, b):
        for t in range(ND):
            pltpu.make_async_copy(table_hbm.at[idx_v.at[g * ND + t, 0]],
                                  rows_v.at[b, t], sems[b]).wait()

    def mac_group(g, b):
        def body(pp, carry):
            mac_point(g * KG + pp, b, pp)
            return carry

        lax.fori_loop(0, KG, body, 0)

    IR = CH * NJR // DIDX

    def chunk_body(ci, carry):
        cb = base + ci * CH
        pltpu.sync_copy(idx_hbm.at[pl.ds((wid * PTS_W // CH + ci) * IR, IR)],
                        idx_v)
        pltpu.sync_copy(wgt_hbm.at[pl.ds(cb, CH)], wgt_v)
        pltpu.sync_copy(pe_hbm.at[pl.ds(cb, CH)], acc_v)
        fire(0, 0)
        fire(1, 1)

        def gpair(i, carry2):
            g0 = 2 * i
            drain(g0, 0)
            mac_group(g0, 0)

            @pl.when(g0 + 2 < NGC)
            def _():
                fire(g0 + 2, 0)

            drain(g0 + 1, 1)
            mac_group(g0 + 1, 1)

            @pl.when(g0 + 3 < NGC)
            def _():
                fire(g0 + 3, 1)

            return carry2

        lax.fori_loop(0, NGC // 2, gpair, 0)
        pltpu.sync_copy(acc_v, out_hbm.at[pl.ds(cb, CH)])
        return carry

    lax.fori_loop(0, PTS_W // CH, chunk_body, 0)


def _run_sc_gather(idx, wgt, pe, table):
    mesh = plsc.VectorSubcoreMesh(core_axis_name="c", subcore_axis_name="s")
    fn = functools.partial(
        pl.kernel,
        mesh=mesh,
        out_type=jax.ShapeDtypeStruct((QPP, C), _F32),
        scratch_types=[
            pltpu.VMEM((CH * NJR // DIDX, 1, DIDX), jnp.int32),
            pltpu.VMEM((CH, NJ), _F32),
            pltpu.VMEM((CH, C), _F32),
            pltpu.VMEM((2, ND, DIDX, 2 * C), jnp.int32),
        ] + [pltpu.SemaphoreType.DMA] * 2,
    )(_sc_body)
    return fn(idx, wgt, pe, table)


def _hmlp_body(x_ref, w1_ref, b1_ref, w2_ref, b2_ref, w3_ref, b3_ref,
               w4_ref, b4_ref, o_ref):
    x = x_ref[...]
    h = jnp.maximum(
        lax.dot_general(x, w1_ref[...], (((1,), (0,)), ((), ())),
                        precision=None) + b1_ref[...], 0.0)
    h = jnp.maximum(
        lax.dot_general(h, w2_ref[...], (((1,), (0,)), ((), ())),
                        precision=None) + b2_ref[...], 0.0)
    h = jnp.maximum(
        lax.dot_general(h, w3_ref[...], (((1,), (0,)), ((), ())),
                        precision=None) + b3_ref[...], 0.0)
    o_ref[...] = lax.dot_general(h, w4_ref[...], (((1,), (0,)), ((), ())),
                                 precision=None) + b4_ref[...]


NQP = 2560
BLK_C = 512


def _run_hmlp(x, w1, b1, w2, b2, w3, b3, w4, b4):
    full = lambda shape: pl.BlockSpec(shape, lambda i: (0,) * len(shape))
    return pl.pallas_call(
        _hmlp_body,
        grid=(NQP // BLK_C,),
        in_specs=[
            pl.BlockSpec((BLK_C, NZ * C), lambda i: (i, 0)),
            full((NZ * C, 1024)), full((1, 1024)),
            full((1024, 1024)), full((1, 1024)),
            full((1024, 1024)), full((1, 1024)),
            full((1024, C)), full((1, C)),
        ],
        out_specs=pl.BlockSpec((BLK_C, C), lambda i: (i, 0)),
        out_shape=jax.ShapeDtypeStruct((NQP, C), _F32),
    )(x, w1, b1, w2, b2, w3, b3, w4, b4)


def _flatten_feats(feat0, feat1, feat2, feat3):
    rows = []
    for f in (feat0, feat1, feat2, feat3):
        g = jnp.transpose(f[0], (0, 2, 3, 1))
        v, h, w, c = g.shape
        pr = jnp.concatenate(
            [g[:, :-1, :-1, :], g[:, :-1, 1:, :],
             g[:, 1:, :-1, :], g[:, 1:, 1:, :]], axis=-1)
        rows.append(pr.reshape(v * (h - 1) * (w - 1), 4 * c))
    t = jnp.concatenate(rows, axis=0)
    t32 = t.reshape(N_ROWS, 16, 32)
    t = jnp.stack([t32[..., :16], t32[..., 16:]], axis=-1).reshape(N_ROWS, 4 * C)
    t16 = t.astype(jnp.bfloat16).reshape(N_ROWS, 2 * C, 2)
    return jax.lax.bitcast_convert_type(t16, jnp.int32)


def kernel(feat0, feat1, feat2, feat3, reference_points, lidar2img,
           pe_w1, pe_b1, pe_w2, pe_b2, wgt_w, wgt_b,
           hm_w1, hm_b1, hm_w2, hm_b2, hm_w3, hm_b3, hm_w4, hm_b4):
    pos = jnp.transpose(reference_points[0], (1, 0, 2)).reshape(QP, 3)
    pos_p = jnp.pad(pos, ((0, QPP - QP), (0, 0)))
    m = lidar2img[0, 0]
    a0 = jnp.repeat(m[:, 0, :], NL, axis=0).T
    a1 = jnp.repeat(m[:, 1, :], NL, axis=0).T
    a2 = jnp.repeat(m[:, 2, :], NL, axis=0).T
    table = _flatten_feats(feat0, feat1, feat2, feat3)

    idx, wgt, pe = _run_precomp(
        pos_p, a0, a1, a2,
        pe_w1, pe_b1.reshape(1, -1), pe_w2, pe_b2.reshape(1, -1),
        wgt_w, wgt_b.reshape(1, -1))

    idx = idx.reshape(QPP * NJR // DIDX, 1, DIDX)
    sampled = _run_sc_gather(idx, wgt, pe, table)

    x = sampled[:QP].reshape(NQ, NZ * C)
    x = jnp.pad(x, ((0, NQP - NQ), (0, 0)))
    out = _run_hmlp(
        x, hm_w1, hm_b1.reshape(1, -1), hm_w2, hm_b2.reshape(1, -1),
        hm_w3, hm_b3.reshape(1, -1), hm_w4, hm_b4.reshape(1, -1))
    return out[:NQ].reshape(1, NQ, C)
